# baseline (speedup 1.0000x reference)
import functools
import math

import jax
import jax.numpy as jnp
from jax import lax
from jax.experimental import pallas as pl
from jax.experimental.pallas import tpu as pltpu
from jax.experimental.pallas import tpu_sc as plsc

_EMB = 64
_SCALE = math.sqrt(_EMB)
_LANES = 16
_CHUNK = 128
_NRING = 3


def _emb_kernel_body(n_per_w, num_cores, tokens_hbm, table_hbm, out_hbm,
                     tok_v, pidx_v, rows_v, out_v, tsems, gsems, ssems):
    n_chunks = n_per_w // _CHUNK
    last = n_chunks - 1
    wid = lax.axis_index("s") * num_cores + lax.axis_index("c")
    base = wid * n_per_w

    def tok_desc(c, r):
        return (tokens_hbm.at[pl.ds(base + c * _CHUNK, _CHUNK)], tok_v.at[r],
                tsems[r])

    def gather_desc(r):
        return (table_hbm.at[pidx_v.at[r]], rows_v.at[r], gsems[r])

    def scatter_desc(c, r):
        return (out_v.at[r], out_hbm.at[pl.ds(base + c * _CHUNK, _CHUNK)],
                ssems[r])

    def compute_pidx(r):
        for m in range(_CHUNK // _LANES):
            sl = pl.ds(m * _LANES, _LANES)
            pidx_v[r, sl] = lax.shift_right_logical(tok_v[r, sl], 1)

    def select_scale(r):
        @plsc.parallel_loop(0, _CHUNK // _LANES)
        def _sel(g):
            hvec = lax.mul(
                lax.bitwise_and(tok_v[r, pl.ds(g * _LANES, _LANES)], 1), 64)
            for i in range(_LANES):
                row = g * _LANES + i
                h64 = hvec[i]
                for d in range(_EMB // _LANES):
                    out_v[r, row, pl.ds(d * _LANES, _LANES)] = (
                        rows_v[r, row, pl.ds(h64 + d * _LANES, _LANES)]
                        * _SCALE)

    def step(c, rc, with_a, with_b, with_d_drain):
        r1 = (rc + 1) % _NRING
        r2 = (rc + 2) % _NRING
        if with_a:
            pltpu.async_copy(*tok_desc(c + 2, r2))
        if with_b:
            pltpu.make_async_copy(*tok_desc(c + 1, r1)).wait()
            compute_pidx(r1)
            pltpu.async_copy(*gather_desc(r1))
        pltpu.make_async_copy(*gather_desc(rc)).wait()
        if with_d_drain:
            pltpu.make_async_copy(*scatter_desc(c - _NRING, rc)).wait()
        select_scale(rc)
        pltpu.async_copy(*scatter_desc(c, rc))

    for c in range(2):
        pltpu.async_copy(*tok_desc(c, c))
        pltpu.make_async_copy(*tok_desc(c, c)).wait()
        compute_pidx(c)
        pltpu.async_copy(*gather_desc(c))

    step(0, 0, True, False, False)
    step(1, 1, True, True, False)
    step(2, 2, True, True, False)

    @pl.loop(_NRING, n_chunks - 2, step=_NRING)
    def _main(i):
        for j in range(_NRING):
            step(i + j, j, True, True, True)

    step(n_chunks - 2, (n_chunks - 2) % _NRING, False, True, True)
    step(n_chunks - 1, (n_chunks - 1) % _NRING, False, False, True)

    for c in range(n_chunks - _NRING, n_chunks):
        pltpu.make_async_copy(*scatter_desc(c, c % _NRING)).wait()


def kernel(tokens, table):
    b, l = tokens.shape
    v, d = table.shape
    n = b * l
    info = plsc.get_sparse_core_info()
    nw = info.num_cores * info.num_subcores
    n_per_w = n // nw

    mesh = plsc.VectorSubcoreMesh(core_axis_name="c", subcore_axis_name="s")
    emb = pl.kernel(
        functools.partial(_emb_kernel_body, n_per_w, info.num_cores),
        out_type=jax.ShapeDtypeStruct((n, d), jnp.float32),
        mesh=mesh,
        scratch_types=[
            pltpu.VMEM((_NRING, _CHUNK), jnp.int32),
            pltpu.VMEM((_NRING, _CHUNK), jnp.int32),
            pltpu.VMEM((_NRING, _CHUNK, 2 * d), jnp.float32),
            pltpu.VMEM((_NRING, _CHUNK, d), jnp.float32),
            [pltpu.SemaphoreType.DMA] * _NRING,
            [pltpu.SemaphoreType.DMA] * _NRING,
            [pltpu.SemaphoreType.DMA] * _NRING,
        ],
        compiler_params=pltpu.CompilerParams(use_tc_tiling_on_sc=True),
    )
    flat = emb(jnp.reshape(tokens, (n,)),
               jnp.reshape(table, (v // 2, 2 * d)))
    return jnp.reshape(flat, (b, l, d))


# speedup vs baseline: 1.0527x; 1.0527x over previous
import functools
import math

import jax
import jax.numpy as jnp
from jax import lax
from jax.experimental import pallas as pl
from jax.experimental.pallas import tpu as pltpu
from jax.experimental.pallas import tpu_sc as plsc

_EMB = 64
_SCALE = math.sqrt(_EMB)
_LANES = 16
_CHUNK = 128
_NRING = 4
_NOUT = 2


def _emb_kernel_body(n_per_w, num_cores, tokens_hbm, table_hbm, out_hbm,
                     tok_v, pidx_v, rows_v, out_v, tsems, gsems, ssems):
    n_chunks = n_per_w // _CHUNK
    wid = lax.axis_index("s") * num_cores + lax.axis_index("c")
    base = wid * n_per_w

    def tok_desc(c, r):
        return (tokens_hbm.at[pl.ds(base + c * _CHUNK, _CHUNK)], tok_v.at[r],
                tsems[r])

    def gather_desc(r):
        return (table_hbm.at[pidx_v.at[r]], rows_v.at[r], gsems[r])

    def scatter_desc(c, o):
        return (out_v.at[o], out_hbm.at[pl.ds(base + c * _CHUNK, _CHUNK)],
                ssems[o])

    def compute_pidx(r):
        for m in range(_CHUNK // _LANES):
            sl = pl.ds(m * _LANES, _LANES)
            pidx_v[r, sl] = lax.shift_right_logical(tok_v[r, sl], 1)

    def select_scale(r, o):
        @plsc.parallel_loop(0, _CHUNK // _LANES)
        def _sel(g):
            hvec = lax.mul(
                lax.bitwise_and(tok_v[r, pl.ds(g * _LANES, _LANES)], 1), 64)
            for i in range(_LANES):
                row = g * _LANES + i
                h64 = hvec[i]
                for d in range(_EMB // _LANES):
                    out_v[o, row, pl.ds(d * _LANES, _LANES)] = (
                        rows_v[r, row, pl.ds(h64 + d * _LANES, _LANES)]
                        * _SCALE)

    def step(c, rc, ro):
        r2 = (rc + 2) % _NRING
        r3 = (rc + 3) % _NRING

        @pl.when(c + 3 < n_chunks)
        def _prefetch_tokens():
            pltpu.async_copy(*tok_desc(c + 3, r3))

        @pl.when(c + 2 < n_chunks)
        def _fire_gather():
            pltpu.make_async_copy(*tok_desc(c + 2, r2)).wait()
            compute_pidx(r2)
            pltpu.async_copy(*gather_desc(r2))

        pltpu.make_async_copy(*gather_desc(rc)).wait()

        @pl.when(c >= _NOUT)
        def _drain_scatter():
            pltpu.make_async_copy(*scatter_desc(c - _NOUT, ro)).wait()

        select_scale(rc, ro)
        pltpu.async_copy(*scatter_desc(c, ro))

    for c in range(2):
        pltpu.async_copy(*tok_desc(c, c))
        pltpu.make_async_copy(*tok_desc(c, c)).wait()
        compute_pidx(c)
        pltpu.async_copy(*gather_desc(c))
    pltpu.async_copy(*tok_desc(2, 2))

    @pl.loop(0, n_chunks, step=_NRING)
    def _main(i):
        for j in range(_NRING):
            step(i + j, j, j % _NOUT)

    for c in range(n_chunks - _NOUT, n_chunks):
        pltpu.make_async_copy(*scatter_desc(c, c % _NOUT)).wait()


def kernel(tokens, table):
    b, l = tokens.shape
    v, d = table.shape
    n = b * l
    info = plsc.get_sparse_core_info()
    nw = info.num_cores * info.num_subcores
    n_per_w = n // nw

    mesh = plsc.VectorSubcoreMesh(core_axis_name="c", subcore_axis_name="s")
    emb = pl.kernel(
        functools.partial(_emb_kernel_body, n_per_w, info.num_cores),
        out_type=jax.ShapeDtypeStruct((n, d), jnp.float32),
        mesh=mesh,
        scratch_types=[
            pltpu.VMEM((_NRING, _CHUNK), jnp.int32),
            pltpu.VMEM((_NRING, _CHUNK), jnp.int32),
            pltpu.VMEM((_NRING, _CHUNK, 2 * d), jnp.float32),
            pltpu.VMEM((_NOUT, _CHUNK, d), jnp.float32),
            [pltpu.SemaphoreType.DMA] * _NRING,
            [pltpu.SemaphoreType.DMA] * _NRING,
            [pltpu.SemaphoreType.DMA] * _NOUT,
        ],
        compiler_params=pltpu.CompilerParams(use_tc_tiling_on_sc=True),
    )
    flat = emb(jnp.reshape(tokens, (n,)),
               jnp.reshape(table, (v // 2, 2 * d)))
    return jnp.reshape(flat, (b, l, d))
